# Initial kernel scaffold; baseline (speedup 1.0000x reference)
import jax
import jax.numpy as jnp
from jax.experimental import pallas as pl


def kernel(node_features, edge_features, params, edge_index):
    raise NotImplementedError("write your pallas kernel here")


# speedup vs baseline: 2.2047x; 2.2047x over previous
import functools

import jax
import jax.numpy as jnp
from jax import lax
from jax.experimental import pallas as pl
from jax.experimental.pallas import tpu as pltpu
from jax.experimental.pallas import tpu_sc as plsc

_NC = 2
_NS = 16
_NW = _NC * _NS
_L = 16
_CH = 128


def _round_up(v, m):
    return (v + m - 1) // m * m


def _sc_gather_body(tbl_p, tbl_q, src, dst, g_out,
                    idx_s, idx_d, idx_iota, ga, gd, zbuf, acc_sh,
                    *, n_chunks, h):
    c = lax.axis_index("c")
    s = lax.axis_index("s")
    w = c * _NS + s
    reps = n_chunks // _NW
    extra = n_chunks - reps * _NW

    zv = jnp.zeros((_L,), jnp.float32)
    base_iota = lax.iota(jnp.int32, _L)

    def _init_idx(i, carry):
        idx_iota[pl.ds(i * _L, _L)] = base_iota + i * _L
        return carry

    lax.fori_loop(0, _CH // _L, _init_idx, 0)

    def _init_z(i, carry):
        for q in range(2 * h // _L):
            zbuf[i, pl.ds(q * _L, _L)] = zv
        return carry

    lax.fori_loop(0, _CH, _init_z, 0)

    acc = acc_sh.at[pl.ds(pl.multiple_of(s * _CH, _CH), _CH), :]

    def _step(k, carry):
        b = pl.multiple_of((w + k * _NW) * _CH, _CH)
        pltpu.sync_copy(src.at[pl.ds(b, _CH)], idx_s)
        pltpu.sync_copy(dst.at[pl.ds(b, _CH)], idx_d)
        pltpu.sync_copy(tbl_p.at[idx_s], ga)
        pltpu.sync_copy(tbl_q.at[idx_d], gd)
        pltpu.sync_copy(zbuf, acc)
        pltpu.sync_copy(ga, acc.at[idx_iota], add=True)
        pltpu.sync_copy(gd, acc.at[idx_iota], add=True)
        pltpu.sync_copy(acc, g_out.at[pl.ds(b, _CH), :])
        return carry

    lax.fori_loop(0, reps, _step, 0)
    if extra:
        @pl.when(w < extra)
        def _():
            _step(reps, 0)


def _make_sc_gather(n_nodes, n_edges, h):
    n_chunks = n_edges // _CH
    mesh = plsc.VectorSubcoreMesh(core_axis_name="c", subcore_axis_name="s")
    body = functools.partial(_sc_gather_body, n_chunks=n_chunks, h=h)
    return pl.kernel(
        body,
        out_type=jax.ShapeDtypeStruct((n_edges, 2 * h), jnp.float32),
        mesh=mesh,
        scratch_types=[
            pltpu.VMEM((_CH,), jnp.int32),
            pltpu.VMEM((_CH,), jnp.int32),
            pltpu.VMEM((_CH,), jnp.int32),
            pltpu.VMEM((_CH, 2 * h), jnp.float32),
            pltpu.VMEM((_CH, 2 * h), jnp.float32),
            pltpu.VMEM((_CH, 2 * h), jnp.float32),
            pltpu.VMEM_SHARED((_NS * _CH, 2 * h), jnp.float32),
        ],
    )


def _sc_scatter_body(t_in, dst, s_out,
                     idx_d, t_st, zbuf, s_sh, *, n_pad, n_chunks, h):
    c = lax.axis_index("c")
    s = lax.axis_index("s")
    w = c * _NS + s
    rows_per_tile = n_pad // _NS
    reps = n_chunks // _NW
    extra = n_chunks - reps * _NW
    zv = jnp.zeros((_L,), jnp.float32)

    def _zb(i, carry):
        for q in range(2 * h // _L):
            zbuf[i, pl.ds(q * _L, _L)] = zv
        return carry

    lax.fori_loop(0, _CH, _zb, 0)
    base_row = pl.multiple_of(s * rows_per_tile, _CH)
    for j in range(rows_per_tile // _CH):
        pltpu.sync_copy(zbuf, s_sh.at[pl.ds(base_row + j * _CH, _CH), :])
    plsc.subcore_barrier()

    def _step(k, carry):
        b = pl.multiple_of((w + k * _NW) * _CH, _CH)
        pltpu.sync_copy(dst.at[pl.ds(b, _CH)], idx_d)
        pltpu.sync_copy(t_in.at[pl.ds(b, _CH), :], t_st)
        pltpu.sync_copy(t_st, s_sh.at[idx_d], add=True)
        return carry

    lax.fori_loop(0, reps, _step, 0)
    if extra:
        @pl.when(w < extra)
        def _():
            _step(reps, 0)
    plsc.subcore_barrier()
    pltpu.sync_copy(s_sh.at[pl.ds(base_row, rows_per_tile), :],
                    s_out.at[c, pl.ds(base_row, rows_per_tile), :])


def _make_sc_scatter(n_nodes, n_edges, h):
    n_chunks = n_edges // _CH
    n_pad = _round_up(n_nodes, _NS * _CH)
    mesh = plsc.VectorSubcoreMesh(core_axis_name="c", subcore_axis_name="s")
    body = functools.partial(_sc_scatter_body, n_pad=n_pad,
                             n_chunks=n_chunks, h=h)
    return pl.kernel(
        body,
        out_type=jax.ShapeDtypeStruct((_NC, n_pad, 2 * h), jnp.float32),
        mesh=mesh,
        scratch_types=[
            pltpu.VMEM((_CH,), jnp.int32),
            pltpu.VMEM((_CH, 2 * h), jnp.float32),
            pltpu.VMEM((_CH, 2 * h), jnp.float32),
            pltpu.VMEM_SHARED((n_pad, 2 * h), jnp.float32),
        ],
    )


def _sc_deg_body(dst, deg_out, idx_d, ones_b, zbuf, deg_sh,
                 *, n_pad, e_per_sub, chunk):
    c = lax.axis_index("c")
    s = lax.axis_index("s")
    rows_per_tile = n_pad // _NS
    zrows = zbuf.shape[0]

    onehot = jnp.where(lax.iota(jnp.int32, _L) == 0,
                       jnp.float32(1.0), jnp.float32(0.0))
    zv = jnp.zeros((_L,), jnp.float32)

    def _fill(i, carry):
        ones_b[i, :] = onehot
        return carry

    lax.fori_loop(0, chunk, _fill, 0)

    def _zb(i, carry):
        zbuf[i, :] = zv
        return carry

    lax.fori_loop(0, zrows, _zb, 0)
    base_row = pl.multiple_of(s * rows_per_tile, zrows)
    for j in range(rows_per_tile // zrows):
        pltpu.sync_copy(zbuf, deg_sh.at[pl.ds(base_row + j * zrows, zrows), :])
    plsc.subcore_barrier()

    ebase = (c * _NS + s) * e_per_sub

    def _step(k, carry):
        b = pl.multiple_of(ebase + k * chunk, chunk)
        pltpu.sync_copy(dst.at[pl.ds(b, chunk)], idx_d)
        pltpu.sync_copy(ones_b, deg_sh.at[idx_d], add=True)
        return carry

    lax.fori_loop(0, e_per_sub // chunk, _step, 0)
    plsc.subcore_barrier()
    pltpu.sync_copy(deg_sh.at[pl.ds(base_row, rows_per_tile), :],
                    deg_out.at[c, pl.ds(base_row, rows_per_tile), :])


def _make_sc_deg(n_nodes, n_edges):
    e_per_sub = n_edges // _NW
    chunk = 80
    zrows = 128
    n_pad = _round_up(n_nodes, _NS * zrows)
    mesh = plsc.VectorSubcoreMesh(core_axis_name="c", subcore_axis_name="s")
    body = functools.partial(_sc_deg_body, n_pad=n_pad,
                             e_per_sub=e_per_sub, chunk=chunk)
    return pl.kernel(
        body,
        out_type=jax.ShapeDtypeStruct((_NC, n_pad, _L), jnp.float32),
        mesh=mesh,
        scratch_types=[
            pltpu.VMEM((chunk,), jnp.int32),
            pltpu.VMEM((chunk, _L), jnp.float32),
            pltpu.VMEM((zrows, _L), jnp.float32),
            pltpu.VMEM_SHARED((n_pad, _L), jnp.float32),
        ],
    )


def _tc_edge_enc_body(ef, v, cv, o0, o1, o2, *, h):
    t = jnp.dot(ef[...], v[...], preferred_element_type=jnp.float32) + cv[...]
    o0[...] = t[:, :h]
    o1[...] = t[:, h:2 * h]
    o2[...] = t[:, 2 * h:]


def _tc_relu_body(g, eb, o, *, h):
    t = jnp.maximum(g[..., :h] + eb[...], 0.0)
    o[...] = jnp.concatenate([t, jnp.zeros_like(t)], axis=1)


def _tc_encode_pre_body(nf, wn, bn, wac, o_x, o_p, o_q, *, h):
    x = jnp.dot(nf[...], wn[...], preferred_element_type=jnp.float32) + bn[...]
    o_x[...] = x
    hac = jnp.dot(x, wac[...], preferred_element_type=jnp.float32)
    z = jnp.zeros_like(hac[:, :h])
    o_p[...] = jnp.concatenate([hac[:, :h], z], axis=1)
    o_q[...] = jnp.concatenate([hac[:, h:], z], axis=1)


def _tc_update_body(x, spa, degp, w2, b2, u1, c1, u2, c2, wac,
                    o_x, o_p=None, o_q=None, *, h, n, has_next):
    ssum = spa[0, :n, :h] + spa[1, :n, :h]
    deg = degp[0, :n, 0] + degp[1, :n, 0]
    agg = (jnp.dot(ssum, w2[...], preferred_element_type=jnp.float32)
           + deg[:, None] * b2[...])
    u1v = u1[...]
    hid = jnp.maximum(
        jnp.dot(x[...], u1v[:h, :], preferred_element_type=jnp.float32)
        + jnp.dot(agg, u1v[h:, :], preferred_element_type=jnp.float32)
        + c1[...], 0.0)
    xo = jnp.dot(hid, u2[...], preferred_element_type=jnp.float32) + c2[...]
    o_x[...] = xo
    if has_next:
        hac = jnp.dot(xo, wac[...], preferred_element_type=jnp.float32)
        z = jnp.zeros_like(hac[:, :h])
        o_p[...] = jnp.concatenate([hac[:, :h], z], axis=1)
        o_q[...] = jnp.concatenate([hac[:, h:], z], axis=1)


def kernel(node_features, edge_features, params, edge_index):
    n, nd = node_features.shape
    e, ed = edge_features.shape
    wn, bn = params['node_enc']
    we, be = params['edge_enc']
    h = wn.shape[1]
    msg = params['msg']
    upd = params['upd']
    nl = len(msg)
    f32 = jnp.float32

    vs, cs, wacs = [], [], []
    for l in range(nl):
        w1, b1, _, _ = msg[l]
        w1b = w1[h:2 * h]
        vs.append(we @ w1b)
        cs.append(be @ w1b + b1)
        wacs.append(jnp.concatenate([w1[:h], w1[2 * h:]], axis=1))
    vmat = jnp.concatenate(vs, axis=1)
    cvec = jnp.concatenate(cs)[None, :]

    src = edge_index[0].astype(jnp.int32)
    dst = edge_index[1].astype(jnp.int32)

    be_rows = 8000
    eb_list = pl.pallas_call(
        functools.partial(_tc_edge_enc_body, h=h),
        grid=(e // be_rows,),
        in_specs=[
            pl.BlockSpec((be_rows, ed), lambda i: (i, 0)),
            pl.BlockSpec((ed, nl * h), lambda i: (0, 0)),
            pl.BlockSpec((1, nl * h), lambda i: (0, 0)),
        ],
        out_specs=[pl.BlockSpec((be_rows, h), lambda i: (i, 0))] * nl,
        out_shape=[jax.ShapeDtypeStruct((e, h), f32)] * nl,
    )(edge_features, vmat, cvec)

    deg_parts = _make_sc_deg(n, e)(dst)

    x, tp, tq = pl.pallas_call(
        functools.partial(_tc_encode_pre_body, h=h),
        out_shape=[jax.ShapeDtypeStruct((n, h), f32),
                   jax.ShapeDtypeStruct((n, 2 * h), f32),
                   jax.ShapeDtypeStruct((n, 2 * h), f32)],
    )(node_features, wn, bn.reshape(1, h), wacs[0])

    sc_gather = _make_sc_gather(n, e, h)
    sc_scatter = _make_sc_scatter(n, e, h)
    relu_rows = 8000
    relu_call = pl.pallas_call(
        functools.partial(_tc_relu_body, h=h),
        grid=(e // relu_rows,),
        in_specs=[
            pl.BlockSpec((relu_rows, 2 * h), lambda i: (i, 0)),
            pl.BlockSpec((relu_rows, h), lambda i: (i, 0)),
        ],
        out_specs=pl.BlockSpec((relu_rows, 2 * h), lambda i: (i, 0)),
        out_shape=jax.ShapeDtypeStruct((e, 2 * h), f32),
    )
    for l in range(nl):
        g = sc_gather(tp, tq, src, dst)
        t = relu_call(g, eb_list[l])
        s_parts = sc_scatter(t, dst)
        _, _, w2, b2 = msg[l]
        u1, c1, u2, c2 = upd[l]
        has_next = l + 1 < nl
        wac_next = wacs[l + 1] if has_next else wacs[0]
        out_shapes = [jax.ShapeDtypeStruct((n, h), f32)]
        if has_next:
            out_shapes += [jax.ShapeDtypeStruct((n, 2 * h), f32)] * 2
        outs = pl.pallas_call(
            functools.partial(_tc_update_body, h=h, n=n, has_next=has_next),
            out_shape=out_shapes,
        )(x, s_parts, deg_parts, w2, b2.reshape(1, h),
          u1, c1.reshape(1, h), u2, c2.reshape(1, h), wac_next)
        if has_next:
            x, tp, tq = outs
        else:
            x = outs[0]
    return x


# speedup vs baseline: 2.6287x; 1.1923x over previous
import functools

import jax
import jax.numpy as jnp
from jax import lax
from jax.experimental import pallas as pl
from jax.experimental.pallas import tpu as pltpu
from jax.experimental.pallas import tpu_sc as plsc

_NC = 2
_NS = 16
_NW = _NC * _NS
_L = 16
_CH = 128


def _round_up(v, m):
    return (v + m - 1) // m * m


def _sc_gather_body(tbl_p, tbl_q, src, dst, g_out,
                    idx_s, idx_d, idx_iota, ga, gd, acc_sh, sem_a, sem_b,
                    *, n_chunks, h):
    c = lax.axis_index("c")
    s = lax.axis_index("s")
    w = c * _NS + s
    reps = n_chunks // _NW
    extra = n_chunks - reps * _NW

    base_iota = lax.iota(jnp.int32, _L)

    def _init_idx(i, carry):
        idx_iota[pl.ds(i * _L, _L)] = base_iota + i * _L
        return carry

    lax.fori_loop(0, _CH // _L, _init_idx, 0)

    acc = acc_sh.at[pl.ds(pl.multiple_of(s * _CH, _CH), _CH), :]

    def _step(k, carry):
        b = pl.multiple_of((w + k * _NW) * _CH, _CH)
        pltpu.sync_copy(src.at[pl.ds(b, _CH)], idx_s)
        pltpu.sync_copy(dst.at[pl.ds(b, _CH)], idx_d)
        da = pltpu.async_copy(tbl_p.at[idx_s], ga, sem_a)
        db = pltpu.async_copy(tbl_q.at[idx_d], gd, sem_b)
        da.wait()
        pltpu.sync_copy(ga, acc.at[idx_iota])
        db.wait()
        pltpu.sync_copy(gd, acc.at[idx_iota], add=True)
        pltpu.sync_copy(acc, g_out.at[pl.ds(b, _CH), :])
        return carry

    lax.fori_loop(0, reps, _step, 0)
    if extra:
        @pl.when(w < extra)
        def _():
            _step(reps, 0)


def _make_sc_gather(n_nodes, n_edges, h):
    n_chunks = n_edges // _CH
    mesh = plsc.VectorSubcoreMesh(core_axis_name="c", subcore_axis_name="s")
    body = functools.partial(_sc_gather_body, n_chunks=n_chunks, h=h)
    return pl.kernel(
        body,
        out_type=jax.ShapeDtypeStruct((n_edges, 2 * h), jnp.float32),
        mesh=mesh,
        scratch_types=[
            pltpu.VMEM((_CH,), jnp.int32),
            pltpu.VMEM((_CH,), jnp.int32),
            pltpu.VMEM((_CH,), jnp.int32),
            pltpu.VMEM((_CH, 2 * h), jnp.float32),
            pltpu.VMEM((_CH, 2 * h), jnp.float32),
            pltpu.VMEM_SHARED((_NS * _CH, 2 * h), jnp.float32),
            pltpu.SemaphoreType.DMA,
            pltpu.SemaphoreType.DMA,
        ],
    )


def _sc_scatter_body(t_in, dst, s_out,
                     idx_d, t_st, zbuf, s_sh, sem_t, *, n_pad, n_chunks, h):
    c = lax.axis_index("c")
    s = lax.axis_index("s")
    w = c * _NS + s
    rows_per_tile = n_pad // _NS
    reps = n_chunks // _NW
    extra = n_chunks - reps * _NW
    zv = jnp.zeros((_L,), jnp.float32)

    def _zb(i, carry):
        for q in range(2 * h // _L):
            zbuf[i, pl.ds(q * _L, _L)] = zv
        return carry

    lax.fori_loop(0, _CH, _zb, 0)
    base_row = pl.multiple_of(s * rows_per_tile, _CH)
    for j in range(rows_per_tile // _CH):
        pltpu.sync_copy(zbuf, s_sh.at[pl.ds(base_row + j * _CH, _CH), :])
    plsc.subcore_barrier()

    def _step(k, carry):
        b = pl.multiple_of((w + k * _NW) * _CH, _CH)
        dt = pltpu.async_copy(t_in.at[pl.ds(b, _CH), :], t_st, sem_t)
        pltpu.sync_copy(dst.at[pl.ds(b, _CH)], idx_d)
        dt.wait()
        pltpu.sync_copy(t_st, s_sh.at[idx_d], add=True)
        return carry

    lax.fori_loop(0, reps, _step, 0)
    if extra:
        @pl.when(w < extra)
        def _():
            _step(reps, 0)
    plsc.subcore_barrier()
    pltpu.sync_copy(s_sh.at[pl.ds(base_row, rows_per_tile), :],
                    s_out.at[c, pl.ds(base_row, rows_per_tile), :])


def _make_sc_scatter(n_nodes, n_edges, h):
    n_chunks = n_edges // _CH
    n_pad = _round_up(n_nodes, _NS * _CH)
    mesh = plsc.VectorSubcoreMesh(core_axis_name="c", subcore_axis_name="s")
    body = functools.partial(_sc_scatter_body, n_pad=n_pad,
                             n_chunks=n_chunks, h=h)
    return pl.kernel(
        body,
        out_type=jax.ShapeDtypeStruct((_NC, n_pad, 2 * h), jnp.float32),
        mesh=mesh,
        scratch_types=[
            pltpu.VMEM((_CH,), jnp.int32),
            pltpu.VMEM((_CH, 2 * h), jnp.float32),
            pltpu.VMEM((_CH, 2 * h), jnp.float32),
            pltpu.VMEM_SHARED((n_pad, 2 * h), jnp.float32),
            pltpu.SemaphoreType.DMA,
        ],
    )


def _sc_deg_body(dst, deg_out, idx_d, ones_b, zbuf, deg_sh,
                 *, n_pad, e_per_sub, chunk):
    c = lax.axis_index("c")
    s = lax.axis_index("s")
    rows_per_tile = n_pad // _NS
    zrows = zbuf.shape[0]

    onehot = jnp.where(lax.iota(jnp.int32, _L) == 0,
                       jnp.float32(1.0), jnp.float32(0.0))
    zv = jnp.zeros((_L,), jnp.float32)

    def _fill(i, carry):
        ones_b[i, :] = onehot
        return carry

    lax.fori_loop(0, chunk, _fill, 0)

    def _zb(i, carry):
        zbuf[i, :] = zv
        return carry

    lax.fori_loop(0, zrows, _zb, 0)
    base_row = pl.multiple_of(s * rows_per_tile, zrows)
    for j in range(rows_per_tile // zrows):
        pltpu.sync_copy(zbuf, deg_sh.at[pl.ds(base_row + j * zrows, zrows), :])
    plsc.subcore_barrier()

    ebase = (c * _NS + s) * e_per_sub

    def _step(k, carry):
        b = pl.multiple_of(ebase + k * chunk, chunk)
        pltpu.sync_copy(dst.at[pl.ds(b, chunk)], idx_d)
        pltpu.sync_copy(ones_b, deg_sh.at[idx_d], add=True)
        return carry

    lax.fori_loop(0, e_per_sub // chunk, _step, 0)
    plsc.subcore_barrier()
    pltpu.sync_copy(deg_sh.at[pl.ds(base_row, rows_per_tile), :],
                    deg_out.at[c, pl.ds(base_row, rows_per_tile), :])


def _make_sc_deg(n_nodes, n_edges):
    e_per_sub = n_edges // _NW
    chunk = 80
    zrows = 128
    n_pad = _round_up(n_nodes, _NS * zrows)
    mesh = plsc.VectorSubcoreMesh(core_axis_name="c", subcore_axis_name="s")
    body = functools.partial(_sc_deg_body, n_pad=n_pad,
                             e_per_sub=e_per_sub, chunk=chunk)
    return pl.kernel(
        body,
        out_type=jax.ShapeDtypeStruct((_NC, n_pad, _L), jnp.float32),
        mesh=mesh,
        scratch_types=[
            pltpu.VMEM((chunk,), jnp.int32),
            pltpu.VMEM((chunk, _L), jnp.float32),
            pltpu.VMEM((zrows, _L), jnp.float32),
            pltpu.VMEM_SHARED((n_pad, _L), jnp.float32),
        ],
    )


def _tc_edge_enc_body(ef, v, cv, o0, o1, o2, *, h):
    t = jnp.dot(ef[...], v[...], preferred_element_type=jnp.float32) + cv[...]
    o0[...] = t[:, :h]
    o1[...] = t[:, h:2 * h]
    o2[...] = t[:, 2 * h:]


def _tc_relu_body(g, eb, o, *, h):
    t = jnp.maximum(g[..., :h] + eb[...], 0.0)
    o[...] = jnp.concatenate([t, jnp.zeros_like(t)], axis=1)


def _tc_encode_pre_body(nf, wn, bn, wac, o_x, o_p, o_q, *, h):
    x = jnp.dot(nf[...], wn[...], preferred_element_type=jnp.float32) + bn[...]
    o_x[...] = x
    hac = jnp.dot(x, wac[...], preferred_element_type=jnp.float32)
    z = jnp.zeros_like(hac[:, :h])
    o_p[...] = jnp.concatenate([hac[:, :h], z], axis=1)
    o_q[...] = jnp.concatenate([hac[:, h:], z], axis=1)


def _tc_update_body(x, spa, degp, w2, b2, u1, c1, u2, c2, wac,
                    o_x, o_p=None, o_q=None, *, h, n, has_next):
    ssum = spa[0, :n, :h] + spa[1, :n, :h]
    deg = degp[0, :n, 0] + degp[1, :n, 0]
    agg = (jnp.dot(ssum, w2[...], preferred_element_type=jnp.float32)
           + deg[:, None] * b2[...])
    u1v = u1[...]
    hid = jnp.maximum(
        jnp.dot(x[...], u1v[:h, :], preferred_element_type=jnp.float32)
        + jnp.dot(agg, u1v[h:, :], preferred_element_type=jnp.float32)
        + c1[...], 0.0)
    xo = jnp.dot(hid, u2[...], preferred_element_type=jnp.float32) + c2[...]
    o_x[...] = xo
    if has_next:
        hac = jnp.dot(xo, wac[...], preferred_element_type=jnp.float32)
        z = jnp.zeros_like(hac[:, :h])
        o_p[...] = jnp.concatenate([hac[:, :h], z], axis=1)
        o_q[...] = jnp.concatenate([hac[:, h:], z], axis=1)


def kernel(node_features, edge_features, params, edge_index):
    n, nd = node_features.shape
    e, ed = edge_features.shape
    wn, bn = params['node_enc']
    we, be = params['edge_enc']
    h = wn.shape[1]
    msg = params['msg']
    upd = params['upd']
    nl = len(msg)
    f32 = jnp.float32

    vs, cs, wacs = [], [], []
    for l in range(nl):
        w1, b1, _, _ = msg[l]
        w1b = w1[h:2 * h]
        vs.append(we @ w1b)
        cs.append(be @ w1b + b1)
        wacs.append(jnp.concatenate([w1[:h], w1[2 * h:]], axis=1))
    vmat = jnp.concatenate(vs, axis=1)
    cvec = jnp.concatenate(cs)[None, :]

    src = edge_index[0].astype(jnp.int32)
    dst = edge_index[1].astype(jnp.int32)

    be_rows = 8000
    eb_list = pl.pallas_call(
        functools.partial(_tc_edge_enc_body, h=h),
        grid=(e // be_rows,),
        in_specs=[
            pl.BlockSpec((be_rows, ed), lambda i: (i, 0)),
            pl.BlockSpec((ed, nl * h), lambda i: (0, 0)),
            pl.BlockSpec((1, nl * h), lambda i: (0, 0)),
        ],
        out_specs=[pl.BlockSpec((be_rows, h), lambda i: (i, 0))] * nl,
        out_shape=[jax.ShapeDtypeStruct((e, h), f32)] * nl,
    )(edge_features, vmat, cvec)

    deg_parts = _make_sc_deg(n, e)(dst)

    x, tp, tq = pl.pallas_call(
        functools.partial(_tc_encode_pre_body, h=h),
        out_shape=[jax.ShapeDtypeStruct((n, h), f32),
                   jax.ShapeDtypeStruct((n, 2 * h), f32),
                   jax.ShapeDtypeStruct((n, 2 * h), f32)],
    )(node_features, wn, bn.reshape(1, h), wacs[0])

    sc_gather = _make_sc_gather(n, e, h)
    sc_scatter = _make_sc_scatter(n, e, h)
    relu_rows = 8000
    relu_call = pl.pallas_call(
        functools.partial(_tc_relu_body, h=h),
        grid=(e // relu_rows,),
        in_specs=[
            pl.BlockSpec((relu_rows, 2 * h), lambda i: (i, 0)),
            pl.BlockSpec((relu_rows, h), lambda i: (i, 0)),
        ],
        out_specs=pl.BlockSpec((relu_rows, 2 * h), lambda i: (i, 0)),
        out_shape=jax.ShapeDtypeStruct((e, 2 * h), f32),
    )
    for l in range(nl):
        g = sc_gather(tp, tq, src, dst)
        t = relu_call(g, eb_list[l])
        s_parts = sc_scatter(t, dst)
        _, _, w2, b2 = msg[l]
        u1, c1, u2, c2 = upd[l]
        has_next = l + 1 < nl
        wac_next = wacs[l + 1] if has_next else wacs[0]
        out_shapes = [jax.ShapeDtypeStruct((n, h), f32)]
        if has_next:
            out_shapes += [jax.ShapeDtypeStruct((n, 2 * h), f32)] * 2
        outs = pl.pallas_call(
            functools.partial(_tc_update_body, h=h, n=n, has_next=has_next),
            out_shape=out_shapes,
        )(x, s_parts, deg_parts, w2, b2.reshape(1, h),
          u1, c1.reshape(1, h), u2, c2.reshape(1, h), wac_next)
        if has_next:
            x, tp, tq = outs
        else:
            x = outs[0]
    return x


# speedup vs baseline: 3.1133x; 1.1843x over previous
import functools

import jax
import jax.numpy as jnp
from jax import lax
from jax.experimental import pallas as pl
from jax.experimental.pallas import tpu as pltpu
from jax.experimental.pallas import tpu_sc as plsc

_NC = 2
_NS = 16
_NW = _NC * _NS
_L = 16
_CH = 128


def _round_up(v, m):
    return (v + m - 1) // m * m


def _sc_gather_body(tbl_p, tbl_q, src, dst, g_out,
                    idx_s, idx_d, idx_s2, idx_d2, idx_iota,
                    ga, gd, ga2, gd2, acc_sh,
                    sem_a, sem_b, sem_a2, sem_b2,
                    *, n_chunks, h):
    c = lax.axis_index("c")
    s = lax.axis_index("s")
    w = c * _NS + s
    reps = n_chunks // _NW
    extra = n_chunks - reps * _NW

    base_iota = lax.iota(jnp.int32, _L)

    def _init_idx(i, carry):
        idx_iota[pl.ds(i * _L, _L)] = base_iota + i * _L
        return carry

    lax.fori_loop(0, _CH // _L, _init_idx, 0)

    acc0 = acc_sh.at[pl.ds(pl.multiple_of(s * 2 * _CH, _CH), _CH), :]
    acc1 = acc_sh.at[pl.ds(pl.multiple_of(s * 2 * _CH + _CH, _CH), _CH), :]

    def _issue(k, islot):
        idx_s, idx_d, ga, gd, sem_a, sem_b = islot
        b = pl.multiple_of((w + k * _NW) * _CH, _CH)
        pltpu.sync_copy(src.at[pl.ds(b, _CH)], idx_s)
        pltpu.sync_copy(dst.at[pl.ds(b, _CH)], idx_d)
        da = pltpu.async_copy(tbl_p.at[idx_s], ga, sem_a)
        db = pltpu.async_copy(tbl_q.at[idx_d], gd, sem_b)
        return b, da, db

    def _finish(binfo, islot, acc):
        idx_s, idx_d, ga, gd, _, _ = islot
        b, da, db = binfo
        da.wait()
        pltpu.sync_copy(ga, acc.at[idx_iota])
        db.wait()
        pltpu.sync_copy(gd, acc.at[idx_iota], add=True)
        pltpu.sync_copy(acc, g_out.at[pl.ds(b, _CH), :])

    slot0 = (idx_s, idx_d, ga, gd, sem_a, sem_b)
    slot1 = (idx_s2, idx_d2, ga2, gd2, sem_a2, sem_b2)

    def _step2(k2, carry):
        i0 = _issue(2 * k2, slot0)
        i1 = _issue(2 * k2 + 1, slot1)
        _finish(i0, slot0, acc0)
        _finish(i1, slot1, acc1)
        return carry

    lax.fori_loop(0, reps // 2, _step2, 0)
    if reps % 2:
        _finish(_issue(reps - 1, slot0), slot0, acc0)
    if extra:
        @pl.when(w < extra)
        def _():
            _finish(_issue(reps, slot0), slot0, acc0)


def _make_sc_gather(n_nodes, n_edges, h):
    n_chunks = n_edges // _CH
    mesh = plsc.VectorSubcoreMesh(core_axis_name="c", subcore_axis_name="s")
    body = functools.partial(_sc_gather_body, n_chunks=n_chunks, h=h)
    return pl.kernel(
        body,
        out_type=jax.ShapeDtypeStruct((n_edges, 2 * h), jnp.float32),
        mesh=mesh,
        scratch_types=[
            pltpu.VMEM((_CH,), jnp.int32),
            pltpu.VMEM((_CH,), jnp.int32),
            pltpu.VMEM((_CH,), jnp.int32),
            pltpu.VMEM((_CH,), jnp.int32),
            pltpu.VMEM((_CH,), jnp.int32),
            pltpu.VMEM((_CH, 2 * h), jnp.float32),
            pltpu.VMEM((_CH, 2 * h), jnp.float32),
            pltpu.VMEM((_CH, 2 * h), jnp.float32),
            pltpu.VMEM((_CH, 2 * h), jnp.float32),
            pltpu.VMEM_SHARED((_NS * 2 * _CH, 2 * h), jnp.float32),
            pltpu.SemaphoreType.DMA,
            pltpu.SemaphoreType.DMA,
            pltpu.SemaphoreType.DMA,
            pltpu.SemaphoreType.DMA,
        ],
    )


def _sc_scatter_body(t_in, dst, s_out,
                     idx_d, idx_d2, t_st, t_st2, s_sh, sem_t, sem_t2,
                     *, n_pad, n_chunks, h):
    c = lax.axis_index("c")
    s = lax.axis_index("s")
    w = c * _NS + s
    rows_per_tile = n_pad // _NS
    reps = n_chunks // _NW
    extra = n_chunks - reps * _NW
    zv = jnp.zeros((_L,), jnp.float32)

    def _zb(i, carry):
        for q in range(2 * h // _L):
            t_st[i, pl.ds(q * _L, _L)] = zv
        return carry

    lax.fori_loop(0, _CH, _zb, 0)
    base_row = pl.multiple_of(s * rows_per_tile, _CH)
    for j in range(rows_per_tile // _CH):
        pltpu.sync_copy(t_st, s_sh.at[pl.ds(base_row + j * _CH, _CH), :])
    plsc.subcore_barrier()

    def _issue(k, islot):
        idxb, tb, semb = islot
        b = pl.multiple_of((w + k * _NW) * _CH, _CH)
        dt = pltpu.async_copy(t_in.at[pl.ds(b, _CH), :], tb, semb)
        pltpu.sync_copy(dst.at[pl.ds(b, _CH)], idxb)
        return dt

    def _finish(dt, islot):
        idxb, tb, _ = islot
        dt.wait()
        pltpu.sync_copy(tb, s_sh.at[idxb], add=True)

    slot0 = (idx_d, t_st, sem_t)
    slot1 = (idx_d2, t_st2, sem_t2)

    def _step2(k2, carry):
        d0 = _issue(2 * k2, slot0)
        d1 = _issue(2 * k2 + 1, slot1)
        _finish(d0, slot0)
        _finish(d1, slot1)
        return carry

    lax.fori_loop(0, reps // 2, _step2, 0)
    if reps % 2:
        _finish(_issue(reps - 1, slot0), slot0)
    if extra:
        @pl.when(w < extra)
        def _():
            _finish(_issue(reps, slot0), slot0)
    plsc.subcore_barrier()
    pltpu.sync_copy(s_sh.at[pl.ds(base_row, rows_per_tile), :],
                    s_out.at[c, pl.ds(base_row, rows_per_tile), :])


def _make_sc_scatter(n_nodes, n_edges, h):
    n_chunks = n_edges // _CH
    n_pad = _round_up(n_nodes, _NS * _CH)
    mesh = plsc.VectorSubcoreMesh(core_axis_name="c", subcore_axis_name="s")
    body = functools.partial(_sc_scatter_body, n_pad=n_pad,
                             n_chunks=n_chunks, h=h)
    return pl.kernel(
        body,
        out_type=jax.ShapeDtypeStruct((_NC, n_pad, 2 * h), jnp.float32),
        mesh=mesh,
        scratch_types=[
            pltpu.VMEM((_CH,), jnp.int32),
            pltpu.VMEM((_CH,), jnp.int32),
            pltpu.VMEM((_CH, 2 * h), jnp.float32),
            pltpu.VMEM((_CH, 2 * h), jnp.float32),
            pltpu.VMEM_SHARED((n_pad, 2 * h), jnp.float32),
            pltpu.SemaphoreType.DMA,
            pltpu.SemaphoreType.DMA,
        ],
    )


def _sc_deg_body(dst, deg_out, idx_d, ones_b, zbuf, deg_sh,
                 *, n_pad, e_per_sub, chunk):
    c = lax.axis_index("c")
    s = lax.axis_index("s")
    rows_per_tile = n_pad // _NS
    zrows = zbuf.shape[0]

    onehot = jnp.where(lax.iota(jnp.int32, _L) == 0,
                       jnp.float32(1.0), jnp.float32(0.0))
    zv = jnp.zeros((_L,), jnp.float32)

    def _fill(i, carry):
        ones_b[i, :] = onehot
        return carry

    lax.fori_loop(0, chunk, _fill, 0)

    def _zb(i, carry):
        zbuf[i, :] = zv
        return carry

    lax.fori_loop(0, zrows, _zb, 0)
    base_row = pl.multiple_of(s * rows_per_tile, zrows)
    for j in range(rows_per_tile // zrows):
        pltpu.sync_copy(zbuf, deg_sh.at[pl.ds(base_row + j * zrows, zrows), :])
    plsc.subcore_barrier()

    ebase = (c * _NS + s) * e_per_sub

    def _step(k, carry):
        b = pl.multiple_of(ebase + k * chunk, chunk)
        pltpu.sync_copy(dst.at[pl.ds(b, chunk)], idx_d)
        pltpu.sync_copy(ones_b, deg_sh.at[idx_d], add=True)
        return carry

    lax.fori_loop(0, e_per_sub // chunk, _step, 0)
    plsc.subcore_barrier()
    pltpu.sync_copy(deg_sh.at[pl.ds(base_row, rows_per_tile), :],
                    deg_out.at[c, pl.ds(base_row, rows_per_tile), :])


def _make_sc_deg(n_nodes, n_edges):
    e_per_sub = n_edges // _NW
    chunk = 80
    zrows = 128
    n_pad = _round_up(n_nodes, _NS * zrows)
    mesh = plsc.VectorSubcoreMesh(core_axis_name="c", subcore_axis_name="s")
    body = functools.partial(_sc_deg_body, n_pad=n_pad,
                             e_per_sub=e_per_sub, chunk=chunk)
    return pl.kernel(
        body,
        out_type=jax.ShapeDtypeStruct((_NC, n_pad, _L), jnp.float32),
        mesh=mesh,
        scratch_types=[
            pltpu.VMEM((chunk,), jnp.int32),
            pltpu.VMEM((chunk, _L), jnp.float32),
            pltpu.VMEM((zrows, _L), jnp.float32),
            pltpu.VMEM_SHARED((n_pad, _L), jnp.float32),
        ],
    )


def _tc_edge_enc_body(ef, v, cv, o0, o1, o2, *, h):
    t = jnp.dot(ef[...], v[...], preferred_element_type=jnp.float32) + cv[...]
    o0[...] = t[:, :h]
    o1[...] = t[:, h:2 * h]
    o2[...] = t[:, 2 * h:]


def _tc_relu_body(g, eb, o, *, h):
    t = jnp.maximum(g[..., :h] + eb[...], 0.0)
    o[...] = jnp.concatenate([t, jnp.zeros_like(t)], axis=1)


def _tc_encode_pre_body(nf, wn, bn, wac, o_x, o_p, o_q, *, h):
    x = jnp.dot(nf[...], wn[...], preferred_element_type=jnp.float32) + bn[...]
    o_x[...] = x
    hac = jnp.dot(x, wac[...], preferred_element_type=jnp.float32)
    z = jnp.zeros_like(hac[:, :h])
    o_p[...] = jnp.concatenate([hac[:, :h], z], axis=1)
    o_q[...] = jnp.concatenate([hac[:, h:], z], axis=1)


def _tc_update_body(x, spa, degp, w2, b2, u1, c1, u2, c2, wac,
                    o_x, o_p=None, o_q=None, *, h, n, has_next):
    ssum = spa[0, :n, :h] + spa[1, :n, :h]
    deg = degp[0, :n, 0] + degp[1, :n, 0]
    agg = (jnp.dot(ssum, w2[...], preferred_element_type=jnp.float32)
           + deg[:, None] * b2[...])
    u1v = u1[...]
    hid = jnp.maximum(
        jnp.dot(x[...], u1v[:h, :], preferred_element_type=jnp.float32)
        + jnp.dot(agg, u1v[h:, :], preferred_element_type=jnp.float32)
        + c1[...], 0.0)
    xo = jnp.dot(hid, u2[...], preferred_element_type=jnp.float32) + c2[...]
    o_x[...] = xo
    if has_next:
        hac = jnp.dot(xo, wac[...], preferred_element_type=jnp.float32)
        z = jnp.zeros_like(hac[:, :h])
        o_p[...] = jnp.concatenate([hac[:, :h], z], axis=1)
        o_q[...] = jnp.concatenate([hac[:, h:], z], axis=1)


def kernel(node_features, edge_features, params, edge_index):
    n, nd = node_features.shape
    e, ed = edge_features.shape
    wn, bn = params['node_enc']
    we, be = params['edge_enc']
    h = wn.shape[1]
    msg = params['msg']
    upd = params['upd']
    nl = len(msg)
    f32 = jnp.float32

    vs, cs, wacs = [], [], []
    for l in range(nl):
        w1, b1, _, _ = msg[l]
        w1b = w1[h:2 * h]
        vs.append(we @ w1b)
        cs.append(be @ w1b + b1)
        wacs.append(jnp.concatenate([w1[:h], w1[2 * h:]], axis=1))
    vmat = jnp.concatenate(vs, axis=1)
    cvec = jnp.concatenate(cs)[None, :]

    src = edge_index[0].astype(jnp.int32)
    dst = edge_index[1].astype(jnp.int32)

    be_rows = 8000
    eb_list = pl.pallas_call(
        functools.partial(_tc_edge_enc_body, h=h),
        grid=(e // be_rows,),
        in_specs=[
            pl.BlockSpec((be_rows, ed), lambda i: (i, 0)),
            pl.BlockSpec((ed, nl * h), lambda i: (0, 0)),
            pl.BlockSpec((1, nl * h), lambda i: (0, 0)),
        ],
        out_specs=[pl.BlockSpec((be_rows, h), lambda i: (i, 0))] * nl,
        out_shape=[jax.ShapeDtypeStruct((e, h), f32)] * nl,
    )(edge_features, vmat, cvec)

    deg_parts = _make_sc_deg(n, e)(dst)

    x, tp, tq = pl.pallas_call(
        functools.partial(_tc_encode_pre_body, h=h),
        out_shape=[jax.ShapeDtypeStruct((n, h), f32),
                   jax.ShapeDtypeStruct((n, 2 * h), f32),
                   jax.ShapeDtypeStruct((n, 2 * h), f32)],
    )(node_features, wn, bn.reshape(1, h), wacs[0])

    sc_gather = _make_sc_gather(n, e, h)
    sc_scatter = _make_sc_scatter(n, e, h)
    relu_rows = 8000
    relu_call = pl.pallas_call(
        functools.partial(_tc_relu_body, h=h),
        grid=(e // relu_rows,),
        in_specs=[
            pl.BlockSpec((relu_rows, 2 * h), lambda i: (i, 0)),
            pl.BlockSpec((relu_rows, h), lambda i: (i, 0)),
        ],
        out_specs=pl.BlockSpec((relu_rows, 2 * h), lambda i: (i, 0)),
        out_shape=jax.ShapeDtypeStruct((e, 2 * h), f32),
    )
    for l in range(nl):
        g = sc_gather(tp, tq, src, dst)
        t = relu_call(g, eb_list[l])
        s_parts = sc_scatter(t, dst)
        _, _, w2, b2 = msg[l]
        u1, c1, u2, c2 = upd[l]
        has_next = l + 1 < nl
        wac_next = wacs[l + 1] if has_next else wacs[0]
        out_shapes = [jax.ShapeDtypeStruct((n, h), f32)]
        if has_next:
            out_shapes += [jax.ShapeDtypeStruct((n, 2 * h), f32)] * 2
        outs = pl.pallas_call(
            functools.partial(_tc_update_body, h=h, n=n, has_next=has_next),
            out_shape=out_shapes,
        )(x, s_parts, deg_parts, w2, b2.reshape(1, h),
          u1, c1.reshape(1, h), u2, c2.reshape(1, h), wac_next)
        if has_next:
            x, tp, tq = outs
        else:
            x = outs[0]
    return x
